# baseline (speedup 1.0000x reference)
import functools

import jax
import jax.numpy as jnp
from jax.experimental import pallas as pl
from jax.experimental.pallas import tpu as pltpu

_K = 100
_FOCAL = 5000.0


def _topk_kernel(logits_ref, scores_ref, qidx_ref, labels_ref, *, num_classes):
    x = logits_ref[...]
    p = jax.nn.sigmoid(x)
    b, nc = p.shape
    col = jax.lax.broadcasted_iota(jnp.int32, (b, nc), 1)
    kcol = jax.lax.broadcasted_iota(jnp.int32, (b, _K), 1)

    def body(i, carry):
        p, sc, qi, lb = carry
        m = jnp.max(p, axis=1, keepdims=True)
        idx = jnp.min(jnp.where(p == m, col, nc), axis=1, keepdims=True)
        sel = kcol == i
        sc = jnp.where(sel, m, sc)
        qi = jnp.where(sel, idx // num_classes, qi)
        lb = jnp.where(sel, idx % num_classes, lb)
        p = jnp.where(col == idx, -1.0, p)
        return p, sc, qi, lb

    sc0 = jnp.zeros((b, _K), jnp.float32)
    qi0 = jnp.zeros((b, _K), jnp.int32)
    lb0 = jnp.zeros((b, _K), jnp.int32)
    _, sc, qi, lb = jax.lax.fori_loop(0, _K, body, (p, sc0, qi0, lb0))
    scores_ref[...] = sc
    qidx_ref[...] = qi
    labels_ref[...] = lb


def _gather_kernel(idx_ref, meta_ref,
                   boxes_ref, pose_ref, beta_ref, expr_ref, cam_ref,
                   kp3d_ref, verts_ref,
                   boxes_out, kp2d_out, pose_out, beta_out, expr_out,
                   cam_out, kp3d_out, verts_out):
    del idx_ref
    b = pl.program_id(0)

    pose_out[...] = pose_ref[...]
    beta_out[...] = beta_ref[...]
    expr_out[...] = expr_ref[...]
    cam_out[...] = cam_ref[...]
    kp3d_out[...] = kp3d_ref[...]
    verts_out[...] = verts_ref[...]

    th = meta_ref[b, 0]
    tw = meta_ref[b, 1]
    cx = boxes_ref[0, 0, 0, 0]
    cy = boxes_ref[0, 0, 0, 1]
    w = boxes_ref[0, 0, 0, 2]
    h = boxes_ref[0, 0, 0, 3]
    x0 = (cx - 0.5 * w) * tw
    y0 = (cy - 0.5 * h) * th
    x1 = (cx + 0.5 * w) * tw
    y1 = (cy + 0.5 * h) * th
    blane = jax.lax.broadcasted_iota(jnp.int32, (1, 1, 1, 4), 3)
    boxes_out[...] = jnp.where(
        blane == 0, x0, jnp.where(blane == 1, y0, jnp.where(blane == 2, x1, y1)))

    ih = meta_ref[b, 2]
    iw = meta_ref[b, 3]
    s = cam_ref[0, 0, 0, 0]
    tx = cam_ref[0, 0, 0, 1]
    ty = cam_ref[0, 0, 0, 2]
    tz = 2.0 * _FOCAL / (iw * s + 1e-9)
    kp = kp3d_ref[...]
    lane = jax.lax.broadcasted_iota(jnp.int32, kp.shape, 3)
    trans = jnp.where(lane == 0, tx, jnp.where(lane == 1, ty, tz))
    pts = kp + trans
    xy = pts[..., 0:2] / (pts[..., 2:3] + 1e-9)
    lane2 = jax.lax.broadcasted_iota(jnp.int32, xy.shape, 3)
    ctr = jnp.where(lane2 == 0, iw * 0.5, ih * 0.5)
    kp2d_out[...] = xy * _FOCAL + ctr


def kernel(pred_logits, pred_boxes, pred_smpl_fullpose, pred_smpl_beta,
           pred_smpl_expr, pred_smpl_cam, pred_smpl_kp3d, pred_smpl_verts,
           target_sizes, img_shape):
    B, N, C = pred_logits.shape
    KP = pred_smpl_kp3d.shape[2]
    V = pred_smpl_verts.shape[2]
    P = pred_smpl_fullpose.shape[2]

    scores, qidx, labels = pl.pallas_call(
        functools.partial(_topk_kernel, num_classes=C),
        out_shape=[
            jax.ShapeDtypeStruct((B, _K), jnp.float32),
            jax.ShapeDtypeStruct((B, _K), jnp.int32),
            jax.ShapeDtypeStruct((B, _K), jnp.int32),
        ],
    )(pred_logits.reshape(B, N * C))

    meta = jnp.concatenate([target_sizes, img_shape], axis=1)

    def row(d):
        return pl.BlockSpec((1, 1, 1, d),
                            lambda bb, kk, idx, mt: (bb, idx[bb, kk], 0, 0))

    def orow(d):
        return pl.BlockSpec((1, 1, 1, d),
                            lambda bb, kk, idx, mt: (bb, kk, 0, 0))

    grid_spec = pltpu.PrefetchScalarGridSpec(
        num_scalar_prefetch=2,
        grid=(B, _K),
        in_specs=[
            row(4),
            row(P),
            row(10),
            row(10),
            row(3),
            pl.BlockSpec((1, 1, KP, 3),
                         lambda bb, kk, idx, mt: (bb, idx[bb, kk], 0, 0)),
            row(V * 3),
        ],
        out_specs=[
            orow(4),
            pl.BlockSpec((1, 1, KP, 2),
                         lambda bb, kk, idx, mt: (bb, kk, 0, 0)),
            orow(P),
            orow(10),
            orow(10),
            orow(3),
            pl.BlockSpec((1, 1, KP, 3),
                         lambda bb, kk, idx, mt: (bb, kk, 0, 0)),
            orow(V * 3),
        ],
    )
    outs = pl.pallas_call(
        _gather_kernel,
        grid_spec=grid_spec,
        out_shape=[
            jax.ShapeDtypeStruct((B, _K, 1, 4), jnp.float32),
            jax.ShapeDtypeStruct((B, _K, KP, 2), jnp.float32),
            jax.ShapeDtypeStruct((B, _K, 1, P), jnp.float32),
            jax.ShapeDtypeStruct((B, _K, 1, 10), jnp.float32),
            jax.ShapeDtypeStruct((B, _K, 1, 10), jnp.float32),
            jax.ShapeDtypeStruct((B, _K, 1, 3), jnp.float32),
            jax.ShapeDtypeStruct((B, _K, KP, 3), jnp.float32),
            jax.ShapeDtypeStruct((B, _K, 1, V * 3), jnp.float32),
        ],
    )(qidx, meta,
      pred_boxes.reshape(B, N, 1, 4),
      pred_smpl_fullpose.reshape(B, N, 1, P),
      pred_smpl_beta.reshape(B, N, 1, 10),
      pred_smpl_expr.reshape(B, N, 1, 10),
      pred_smpl_cam.reshape(B, N, 1, 3),
      pred_smpl_kp3d,
      pred_smpl_verts.reshape(B, N, 1, V * 3))

    boxes, kp2d, pose_o, beta_o, expr_o, cam_o, kp3d_o, verts_o = outs
    return (scores, labels, boxes.reshape(B, _K, 4), kp2d,
            pose_o.reshape(B, _K, P), beta_o.reshape(B, _K, 10),
            expr_o.reshape(B, _K, 10), cam_o.reshape(B, _K, 3),
            kp3d_o, verts_o.reshape(B, _K, V, 3))
